# Initial kernel scaffold; baseline (speedup 1.0000x reference)
import jax
import jax.numpy as jnp
from jax.experimental import pallas as pl


def kernel(x, w1, b1, w2, b2, w3, b3):
    raise NotImplementedError("write your pallas kernel here")


# speedup vs baseline: 1.0420x; 1.0420x over previous
import jax
import jax.numpy as jnp
from jax.experimental import pallas as pl
from jax.experimental.pallas import tpu as pltpu

_NEG_BIG = -1e30


def _mlp_softmax_kernel(x_ref, w1_ref, b1_ref, w2_ref, b2_ref, w3_ref, b3_ref,
                        o_ref):
    x = x_ref[...].astype(jnp.bfloat16)
    h1 = jnp.dot(x, w1_ref[...], preferred_element_type=jnp.float32) + b1_ref[...]
    h1 = jnp.maximum(h1, 0.0).astype(jnp.bfloat16)
    h2 = jnp.dot(h1, w2_ref[...], preferred_element_type=jnp.float32) + b2_ref[...]
    h2 = jnp.maximum(h2, 0.0).astype(jnp.bfloat16)
    z = jnp.dot(h2, w3_ref[...], preferred_element_type=jnp.float32) + b3_ref[...]
    z_max = jnp.max(z, axis=-1, keepdims=True)
    e = jnp.exp(z - z_max)
    denom = jnp.sum(e, axis=-1, keepdims=True)
    p = e / denom
    o_ref[...] = p[:, : o_ref.shape[-1]]


def kernel(x, w1, b1, w2, b2, w3, b3, *, block_b=512):
    B, num_in = x.shape
    num_hidden = w1.shape[1]
    num_out = w3.shape[1]

    out_pad = ((num_out + 127) // 128) * 128
    pad_n = out_pad - num_out
    w3p = jnp.pad(w3, ((0, 0), (0, pad_n))).astype(jnp.bfloat16)
    b3p = jnp.pad(b3, ((0, 0), (0, pad_n)), constant_values=_NEG_BIG)
    w1h = w1.astype(jnp.bfloat16)
    w2h = w2.astype(jnp.bfloat16)

    nb = pl.cdiv(B, block_b)
    bp = nb * block_b
    if bp != B:
        x = jnp.pad(x, ((0, bp - B), (0, 0)))

    out = pl.pallas_call(
        _mlp_softmax_kernel,
        out_shape=jax.ShapeDtypeStruct((bp, num_out), jnp.float32),
        grid=(nb,),
        in_specs=[
            pl.BlockSpec((block_b, num_in), lambda i: (i, 0)),
            pl.BlockSpec((num_in, num_hidden), lambda i: (0, 0)),
            pl.BlockSpec((1, num_hidden), lambda i: (0, 0)),
            pl.BlockSpec((num_hidden, num_hidden), lambda i: (0, 0)),
            pl.BlockSpec((1, num_hidden), lambda i: (0, 0)),
            pl.BlockSpec((num_hidden, out_pad), lambda i: (0, 0)),
            pl.BlockSpec((1, out_pad), lambda i: (0, 0)),
        ],
        out_specs=pl.BlockSpec((block_b, num_out), lambda i: (i, 0)),
        compiler_params=pltpu.CompilerParams(
            dimension_semantics=("parallel",)),
    )(x, w1h, b1, w2h, b2, w3p, b3p)
    return out[:B]


# speedup vs baseline: 1.0477x; 1.0055x over previous
import jax
import jax.numpy as jnp
from jax.experimental import pallas as pl
from jax.experimental.pallas import tpu as pltpu

_NEG_BIG = -1e30


def _mlp_softmax_kernel(x_ref, w1_ref, b1_ref, w2_ref, b2_ref, w3_ref, b3_ref,
                        o_ref):
    x = x_ref[...].astype(jnp.bfloat16)
    h1 = jnp.dot(x, w1_ref[...], preferred_element_type=jnp.float32) + b1_ref[...]
    h1 = jnp.maximum(h1, 0.0).astype(jnp.bfloat16)
    h2 = jnp.dot(h1, w2_ref[...], preferred_element_type=jnp.float32) + b2_ref[...]
    h2 = jnp.maximum(h2, 0.0).astype(jnp.bfloat16)
    z = jnp.dot(h2, w3_ref[...], preferred_element_type=jnp.float32) + b3_ref[...]
    z_max = jnp.max(z, axis=-1, keepdims=True)
    e = jnp.exp(z - z_max)
    denom = jnp.sum(e, axis=-1, keepdims=True)
    p = e / denom
    o_ref[...] = p[:, : o_ref.shape[-1]]


def kernel(x, w1, b1, w2, b2, w3, b3, *, block_b=1024):
    B, num_in = x.shape
    num_hidden = w1.shape[1]
    num_out = w3.shape[1]

    out_pad = ((num_out + 127) // 128) * 128
    pad_n = out_pad - num_out
    w3p = jnp.pad(w3, ((0, 0), (0, pad_n))).astype(jnp.bfloat16)
    b3p = jnp.pad(b3, ((0, 0), (0, pad_n)), constant_values=_NEG_BIG)
    w1h = w1.astype(jnp.bfloat16)
    w2h = w2.astype(jnp.bfloat16)

    nb = pl.cdiv(B, block_b)
    bp = nb * block_b
    if bp != B:
        x = jnp.pad(x, ((0, bp - B), (0, 0)))

    out = pl.pallas_call(
        _mlp_softmax_kernel,
        out_shape=jax.ShapeDtypeStruct((bp, num_out), jnp.float32),
        grid=(nb,),
        in_specs=[
            pl.BlockSpec((block_b, num_in), lambda i: (i, 0)),
            pl.BlockSpec((num_in, num_hidden), lambda i: (0, 0)),
            pl.BlockSpec((1, num_hidden), lambda i: (0, 0)),
            pl.BlockSpec((num_hidden, num_hidden), lambda i: (0, 0)),
            pl.BlockSpec((1, num_hidden), lambda i: (0, 0)),
            pl.BlockSpec((num_hidden, out_pad), lambda i: (0, 0)),
            pl.BlockSpec((1, out_pad), lambda i: (0, 0)),
        ],
        out_specs=pl.BlockSpec((block_b, num_out), lambda i: (i, 0)),
        compiler_params=pltpu.CompilerParams(
            dimension_semantics=("parallel",)),
    )(x, w1h, b1, w2h, b2, w3p, b3p)
    return out[:B]


# speedup vs baseline: 1.1081x; 1.0577x over previous
import jax
import jax.numpy as jnp
from jax.experimental import pallas as pl
from jax.experimental.pallas import tpu as pltpu


def _mlp_softmax_kernel(x_ref, w1_ref, b1_ref, w2_ref, b2_ref, w3_ref, b3_ref,
                        o_ref):
    x = x_ref[...].astype(jnp.bfloat16)
    h1 = jnp.dot(x, w1_ref[...].astype(jnp.bfloat16),
                 preferred_element_type=jnp.float32) + b1_ref[...]
    h1 = jnp.maximum(h1, 0.0).astype(jnp.bfloat16)
    h2 = jnp.dot(h1, w2_ref[...].astype(jnp.bfloat16),
                 preferred_element_type=jnp.float32) + b2_ref[...]
    h2 = jnp.maximum(h2, 0.0).astype(jnp.bfloat16)
    z = jnp.dot(h2, w3_ref[...].astype(jnp.bfloat16),
                preferred_element_type=jnp.float32) + b3_ref[...]
    z_max = jnp.max(z, axis=-1, keepdims=True)
    e = jnp.exp(z - z_max)
    denom = jnp.sum(e, axis=-1, keepdims=True)
    o_ref[...] = e / denom


def kernel(x, w1, b1, w2, b2, w3, b3, *, block_b=512):
    B, num_in = x.shape
    num_hidden = w1.shape[1]
    num_out = w3.shape[1]

    nb = pl.cdiv(B, block_b)
    bp = nb * block_b
    if bp != B:
        x = jnp.pad(x, ((0, bp - B), (0, 0)))

    single = pl.Buffered(buffer_count=1)
    out = pl.pallas_call(
        _mlp_softmax_kernel,
        out_shape=jax.ShapeDtypeStruct((bp, num_out), jnp.float32),
        grid=(nb,),
        in_specs=[
            pl.BlockSpec((block_b, num_in), lambda i: (i, 0)),
            pl.BlockSpec((num_in, num_hidden), lambda i: (0, 0),
                         pipeline_mode=single),
            pl.BlockSpec((1, num_hidden), lambda i: (0, 0),
                         pipeline_mode=single),
            pl.BlockSpec((num_hidden, num_hidden), lambda i: (0, 0),
                         pipeline_mode=single),
            pl.BlockSpec((1, num_hidden), lambda i: (0, 0),
                         pipeline_mode=single),
            pl.BlockSpec((num_hidden, num_out), lambda i: (0, 0),
                         pipeline_mode=single),
            pl.BlockSpec((1, num_out), lambda i: (0, 0),
                         pipeline_mode=single),
        ],
        out_specs=pl.BlockSpec((block_b, num_out), lambda i: (i, 0)),
        compiler_params=pltpu.CompilerParams(
            dimension_semantics=("arbitrary",)),
    )(x, w1, b1, w2, b2, w3, b3)
    return out[:B]
